# baseline (speedup 1.0000x reference)
import functools

import jax
import jax.numpy as jnp
import numpy as np
from jax import lax
from jax.experimental import pallas as pl
from jax.experimental.pallas import tpu as pltpu
from jax.experimental.pallas import tpu_sc as plsc

S = 384
BI = 128
BR = 128
BC = 128
_NBITS = 9

_NC = 2
_NS = 16
_L = 16
_IROWS = S // _NS
_CHALF = S // _NC


def _sc_gather_body(x_hbm, dt_hbm, xv, outv, dbl):
    core = lax.axis_index("c")
    sub = lax.axis_index("s")
    c0 = sub * _IROWS

    @pl.when(core == 0)
    def _work():
        pltpu.sync_copy(x_hbm.at[pl.ds(c0, _IROWS), :], xv)
        lane = lax.broadcasted_iota(jnp.int32, (_L,), 0)
        for il in range(_IROWS):
            c = c0 + il
            for j in range(S // _L):
                dbl[pl.ds(c + j * _L, _L)] = xv[il, pl.ds(j * _L, _L)]
            for j in range(S // _L):
                ivec = j * _L + lane
                outv[il, pl.ds(j * _L, _L)] = jnp.where(
                    ivec >= c, dbl[pl.ds(j * _L, _L)], 0.0)
        pltpu.sync_copy(outv, dt_hbm.at[pl.ds(c0, _IROWS), :])


_sc_gather = pl.kernel(
    _sc_gather_body,
    out_type=jax.ShapeDtypeStruct((S, S), jnp.float32),
    mesh=plsc.VectorSubcoreMesh(core_axis_name="c", subcore_axis_name="s",
                                num_cores=_NC, num_subcores=_NS),
    scratch_types=[
        pltpu.VMEM((_IROWS, S), jnp.float32),
        pltpu.VMEM((_IROWS, S), jnp.float32),
        pltpu.VMEM((2 * S, ), jnp.float32),
    ],
)


def _roll_up_cols(a, amounts, s):
    for k in range(_NBITS):
        shift = (1 << k) % s
        if shift == 0:
            continue
        rolled = jnp.concatenate([a[shift:, :], a[:shift, :]], axis=0)
        a = jnp.where(((amounts >> k) & 1) == 1, rolled, a)
    return a


def _valid_triples():
    tri = []
    for ib in range(S // BI):
        imax = ib * BI + BI - 1
        nr = -(-(imax + 1) // BR)
        nc = -(-(imax + 1) // BC)
        for rb in range(nr):
            for cb in range(nc):
                tri.append((ib, rb, cb))
    return np.asarray(tri, dtype=np.int32).T


_TRIPLES = _valid_triples()
_NSTEPS = _TRIPLES.shape[1]


def _fused_body(tri_ref, din_ref, x_ref, b_ref, w_ref, out_ref, d_scr, acc_scr):
    s = pl.program_id(0)
    ib = tri_ref[0, s]
    rb = tri_ref[1, s]
    cb = tri_ref[2, s]
    rows = jax.lax.broadcasted_iota(jnp.int32, (S, S), 0)
    cols = jax.lax.broadcasted_iota(jnp.int32, (S, S), 1)

    @pl.when(s == 0)
    def _stage_d():
        d_scr[...] = din_ref[...].T.reshape(S, 1, S)

    w = w_ref[...]
    d = d_scr[pl.ds(ib * BI, BI), :, pl.ds(cb * BC, BC)]
    contrib = jnp.sum(w * d, axis=-1)

    @pl.when(cb == 0)
    def _init():
        acc_scr[pl.ds(ib * BI, BI), pl.ds(rb * BR, BR)] = contrib

    @pl.when(cb != 0)
    def _acc():
        acc_scr[pl.ds(ib * BI, BI), pl.ds(rb * BR, BR)] += contrib

    @pl.when(s == _NSTEPS - 1)
    def _epilogue():
        t = jnp.where(cols <= rows, acc_scr[...] + b_ref[...], 0.0)
        y = _roll_up_cols(t, cols, S)
        out_ref[...] = jnp.where(rows + cols <= S - 1, y, x_ref[...])


@jax.jit
def kernel(x, W, b):
    x0 = x[0]
    d = _sc_gather(x0)
    y = pl.pallas_call(
        _fused_body,
        grid_spec=pltpu.PrefetchScalarGridSpec(
            num_scalar_prefetch=1,
            grid=(_NSTEPS,),
            in_specs=[
                pl.BlockSpec((S, S), lambda s, t: (0, 0)),
                pl.BlockSpec((S, S), lambda s, t: (0, 0)),
                pl.BlockSpec((S, S), lambda s, t: (0, 0)),
                pl.BlockSpec((BI, BR, BC), lambda s, t: (t[0, s], t[1, s], t[2, s])),
            ],
            out_specs=pl.BlockSpec((S, S), lambda s, t: (0, 0)),
            scratch_shapes=[
                pltpu.VMEM((S, 1, S), jnp.float32),
                pltpu.VMEM((S, S), jnp.float32),
            ],
        ),
        out_shape=jax.ShapeDtypeStruct((S, S), jnp.float32),
    )(jnp.asarray(_TRIPLES), d, x0, b, W)
    return y[None, :, :]


# speedup vs baseline: 1.4557x; 1.4557x over previous
import jax
import jax.numpy as jnp
import numpy as np
from jax.experimental import pallas as pl
from jax.experimental.pallas import tpu as pltpu

S = 384
BI = 128
BR = 128
BC = 128
_NBITS = 9


def _roll_up_cols(a, amounts, s):
    for k in range(_NBITS):
        shift = (1 << k) % s
        if shift == 0:
            continue
        rolled = jnp.concatenate([a[shift:, :], a[:shift, :]], axis=0)
        a = jnp.where(((amounts >> k) & 1) == 1, rolled, a)
    return a


def _roll_down_cols(a, amounts, s):
    for k in range(_NBITS):
        shift = (1 << k) % s
        if shift == 0:
            continue
        rolled = jnp.concatenate([a[s - shift:, :], a[:s - shift, :]], axis=0)
        a = jnp.where(((amounts >> k) & 1) == 1, rolled, a)
    return a


def _valid_triples():
    tri = []
    for ib in range(S // BI):
        imax = ib * BI + BI - 1
        nr = -(-(imax + 1) // BR)
        nc = -(-(imax + 1) // BC)
        for rb in range(nr):
            for cb in range(nc):
                tri.append((ib, rb, cb))
    return np.asarray(tri, dtype=np.int32).T


_TRIPLES = _valid_triples()
_NSTEPS = _TRIPLES.shape[1]


def _fused_body(tri_ref, xt_ref, x_ref, b_ref, w_ref, out_ref, d_scr, acc_scr):
    s = pl.program_id(0)
    ib = tri_ref[0, s]
    rb = tri_ref[1, s]
    cb = tri_ref[2, s]
    rows = jax.lax.broadcasted_iota(jnp.int32, (S, S), 0)
    cols = jax.lax.broadcasted_iota(jnp.int32, (S, S), 1)

    @pl.when(s == 0)
    def _gather():
        d = _roll_down_cols(xt_ref[...], cols, S)
        d_scr[...] = jnp.where(cols <= rows, d, 0.0).reshape(S, 1, S)

    w = w_ref[...]
    d = d_scr[pl.ds(ib * BI, BI), :, pl.ds(cb * BC, BC)]
    prod = w * d
    contrib = jnp.sum(prod, axis=-1)

    @pl.when(cb == 0)
    def _init():
        acc_scr[pl.ds(ib * BI, BI), pl.ds(rb * BR, BR)] = contrib

    @pl.when(cb != 0)
    def _acc():
        acc_scr[pl.ds(ib * BI, BI), pl.ds(rb * BR, BR)] += contrib

    @pl.when(s == _NSTEPS - 1)
    def _epilogue():
        t = jnp.where(cols <= rows, acc_scr[...] + b_ref[...], 0.0)
        y = _roll_up_cols(t, cols, S)
        out_ref[...] = jnp.where(rows + cols <= S - 1, y, x_ref[...])


@jax.jit
def kernel(x, W, b):
    x0 = x[0]
    y = pl.pallas_call(
        _fused_body,
        grid_spec=pltpu.PrefetchScalarGridSpec(
            num_scalar_prefetch=1,
            grid=(_NSTEPS,),
            in_specs=[
                pl.BlockSpec((S, S), lambda s, t: (0, 0)),
                pl.BlockSpec((S, S), lambda s, t: (0, 0)),
                pl.BlockSpec((S, S), lambda s, t: (0, 0)),
                pl.BlockSpec((BI, BR, BC), lambda s, t: (t[0, s], t[1, s], t[2, s])),
            ],
            out_specs=pl.BlockSpec((S, S), lambda s, t: (0, 0)),
            scratch_shapes=[
                pltpu.VMEM((S, 1, S), jnp.float32),
                pltpu.VMEM((S, S), jnp.float32),
            ],
        ),
        out_shape=jax.ShapeDtypeStruct((S, S), jnp.float32),
    )(jnp.asarray(_TRIPLES), x0.T, x0, b, W)
    return y[None, :, :]
